# baseline (speedup 1.0000x reference)
import functools

import jax
import jax.numpy as jnp
from jax import lax
from jax.experimental import pallas as pl
from jax.experimental.pallas import tpu as pltpu
from jax.experimental.pallas import tpu_sc as plsc

N = 10000
E = 320000
D = 128
NG = 16
NC, NS = 2, 16
NW = NC * NS
CHUNK = 128
CPW = 80
EPAD = NW * CPW * CHUNK
NPAD = 10112
ROWS_PT = NPAD // NS
NBLK = NPAD // CHUNK

_MESH = plsc.VectorSubcoreMesh(core_axis_name="c", subcore_axis_name="s")


def _zero_my_slice(acc, zeros_hbm, sid):
    base = sid * ROWS_PT
    nfull = ROWS_PT // CHUNK
    for k in range(nfull):
        pltpu.sync_copy(zeros_hbm, acc.at[pl.ds(base + k * CHUNK, CHUNK)])
    rem = ROWS_PT % CHUNK
    if rem:
        pltpu.sync_copy(zeros_hbm.at[pl.ds(0, rem)],
                        acc.at[pl.ds(base + nfull * CHUNK, rem)])


@functools.partial(
    pl.kernel,
    out_type=jax.ShapeDtypeStruct((NC, NPAD, D), jnp.float32),
    mesh=_MESH,
    scratch_types=[
        pltpu.VMEM_SHARED((NPAD, D), jnp.float32),
        pltpu.VMEM((CPW, CHUNK), jnp.int32),
        pltpu.VMEM((CHUNK, D), jnp.float32),
        pltpu.SemaphoreType.DMA,
    ],
)
def _dega_kernel(dst_hbm, eye_hbm, zeros_hbm, out_hbm, acc, dst_sl, eye_v, ssem):
    cid = lax.axis_index("c")
    sid = lax.axis_index("s")
    wid = sid * NC + cid
    _zero_my_slice(acc, zeros_hbm, sid)
    pltpu.sync_copy(eye_hbm, eye_v)
    pltpu.sync_copy(dst_hbm.at[wid], dst_sl)
    plsc.subcore_barrier()

    def body(j, carry):
        pltpu.async_copy(eye_v, acc.at[dst_sl.at[j]], ssem, add=True)

        @pl.when(j >= 4)
        def _():
            pltpu.make_async_copy(eye_v, acc.at[dst_sl.at[0]], ssem).wait()

        return carry

    lax.fori_loop(0, CPW, body, 0)
    for _ in range(4):
        pltpu.make_async_copy(eye_v, acc.at[dst_sl.at[0]], ssem).wait()
    plsc.subcore_barrier()
    pltpu.sync_copy(acc.at[pl.ds(sid * ROWS_PT, ROWS_PT)],
                    out_hbm.at[cid, pl.ds(sid * ROWS_PT, ROWS_PT)])


TOT_CHUNKS = EPAD // CHUNK
SLAB = CPW // 2


@functools.partial(
    pl.kernel,
    out_type=jax.ShapeDtypeStruct((NC, NPAD, D), jnp.float32),
    mesh=_MESH,
    scratch_types=[
        pltpu.VMEM_SHARED((NPAD, D), jnp.float32),
        pltpu.VMEM((SLAB, CHUNK), jnp.int32),
        pltpu.VMEM((SLAB, CHUNK), jnp.int32),
        [pltpu.VMEM((CHUNK, D), jnp.float32) for _ in range(2)],
        [pltpu.SemaphoreType.DMA for _ in range(2)],
        [pltpu.SemaphoreType.DMA for _ in range(2)],
    ],
)
def _scat_kernel(y_hbm, src_hbm, dst_hbm, zeros_hbm, out_hbm, acc,
                 src_sl, dst_sl, rows, gsem, ssem):
    cid = lax.axis_index("c")
    sid = lax.axis_index("s")
    _zero_my_slice(acc, zeros_hbm, sid)
    plsc.subcore_barrier()

    wid = sid * NC + cid
    base = wid * CPW

    for h in range(2):
        off = pl.multiple_of(base + h * SLAB, 8)
        pltpu.sync_copy(src_hbm.at[pl.ds(off, SLAB)], src_sl)
        pltpu.sync_copy(dst_hbm.at[pl.ds(off, SLAB)], dst_sl)

        def round_body(r, carry):
            for b in range(2):
                j = r * 2 + b

                @pl.when(r > 0)
                def _():
                    pltpu.make_async_copy(rows[b], acc.at[dst_sl.at[j]],
                                          ssem[b]).wait()

                pltpu.async_copy(y_hbm.at[src_sl.at[j]], rows[b], gsem[b])
            for b in range(2):
                j = r * 2 + b
                pltpu.make_async_copy(y_hbm.at[src_sl.at[j]], rows[b],
                                      gsem[b]).wait()
                pltpu.async_copy(rows[b], acc.at[dst_sl.at[j]], ssem[b],
                                 add=True)
            return carry

        lax.fori_loop(0, SLAB // 2, round_body, 0)
        for b in range(2):
            pltpu.make_async_copy(rows[b], acc.at[dst_sl.at[0]],
                                  ssem[b]).wait()
    plsc.subcore_barrier()
    pltpu.sync_copy(acc.at[pl.ds(sid * ROWS_PT, ROWS_PT)],
                    out_hbm.at[cid, pl.ds(sid * ROWS_PT, ROWS_PT)])


def _embed_body(x_ref, we_ref, wg_ref, be_ref, dega_ref, y_ref, dinv_ref):
    w2 = jnp.dot(we_ref[...], wg_ref[...], preferred_element_type=jnp.float32)
    c = jnp.dot(be_ref[...], wg_ref[...], preferred_element_type=jnp.float32)
    hw = jnp.dot(x_ref[...], w2, preferred_element_type=jnp.float32) + c
    deg = jnp.sum(dega_ref[0] + dega_ref[1], axis=1, keepdims=True) + 1.0
    dinv = jnp.broadcast_to(lax.rsqrt(deg), hw.shape)
    y_ref[...] = hw * dinv
    dinv_ref[...] = dinv


def _tc_embed(x_pad, W_embed, W_gcn, b_embed2, dega):
    return pl.pallas_call(
        _embed_body,
        out_shape=(jax.ShapeDtypeStruct((NPAD, D), jnp.float32),
                   jax.ShapeDtypeStruct((NPAD, D), jnp.float32)),
        grid=(NBLK,),
        in_specs=[
            pl.BlockSpec((CHUNK, D), lambda i: (i, 0)),
            pl.BlockSpec((D, D), lambda i: (0, 0)),
            pl.BlockSpec((D, D), lambda i: (0, 0)),
            pl.BlockSpec((1, D), lambda i: (0, 0)),
            pl.BlockSpec((NC, CHUNK, D), lambda i: (0, i, 0)),
        ],
        out_specs=(pl.BlockSpec((CHUNK, D), lambda i: (i, 0)),
                   pl.BlockSpec((CHUNK, D), lambda i: (i, 0))),
    )(x_pad, W_embed, W_gcn, b_embed2, dega)


def _final_body(part_ref, y_ref, dinv_ref, batch_ref, bg_ref, wm_ref, bm_ref,
                out_ref, gsum_ref, cnt_ref):
    i = pl.program_id(0)

    @pl.when(i == 0)
    def _():
        gsum_ref[...] = jnp.zeros_like(gsum_ref)
        cnt_ref[...] = jnp.zeros_like(cnt_ref)

    t = part_ref[0] + part_ref[1] + y_ref[...]
    h2 = jnp.maximum(t * dinv_ref[...] + bg_ref[...], 0.0)
    b = batch_ref[0]
    onehot_t = (lax.broadcasted_iota(jnp.int32, (NG, CHUNK), 0) == b
                ).astype(jnp.float32)
    gsum_ref[...] += jnp.dot(onehot_t, h2, preferred_element_type=jnp.float32)
    cnt_ref[...] = cnt_ref[...] + jnp.sum(onehot_t, axis=1, keepdims=True)

    @pl.when(i == NBLK - 1)
    def _():
        gf = gsum_ref[...] / jnp.maximum(cnt_ref[...], 1.0)
        out_ref[...] = jnp.dot(gf, wm_ref[...],
                               preferred_element_type=jnp.float32) + bm_ref[...]


def _tc_final(part, y, dinvb, batch3, b_gcn2, W_mlp, b_mlp2):
    return pl.pallas_call(
        _final_body,
        out_shape=jax.ShapeDtypeStruct((NG, D), jnp.float32),
        grid=(NBLK,),
        in_specs=[
            pl.BlockSpec((NC, CHUNK, D), lambda i: (0, i, 0)),
            pl.BlockSpec((CHUNK, D), lambda i: (i, 0)),
            pl.BlockSpec((CHUNK, D), lambda i: (i, 0)),
            pl.BlockSpec((1, 1, CHUNK), lambda i: (i, 0, 0)),
            pl.BlockSpec((1, D), lambda i: (0, 0)),
            pl.BlockSpec((D, D), lambda i: (0, 0)),
            pl.BlockSpec((1, D), lambda i: (0, 0)),
        ],
        out_specs=pl.BlockSpec((NG, D), lambda i: (0, 0)),
        scratch_shapes=[
            pltpu.VMEM((NG, D), jnp.float32),
            pltpu.VMEM((NG, 1), jnp.float32),
        ],
    )(part, y, dinvb, batch3, b_gcn2, W_mlp, b_mlp2)


def kernel(x, edge_index, batch, W_embed, b_embed, W_gcn, b_gcn,
           W_aff, b_aff, W_mlp, b_mlp):
    del W_aff, b_aff
    x_pad = jnp.pad(x, ((0, NPAD - N), (0, 0)))
    pad_idx = jnp.full((EPAD - E,), NPAD - 1, dtype=jnp.int32)
    src_r = jnp.concatenate([edge_index[0], pad_idx]).reshape(NW, CPW, CHUNK)
    dst_r = jnp.concatenate([edge_index[1], pad_idx]).reshape(NW, CPW, CHUNK)
    pad_sl = jnp.full((SLAB, CHUNK), NPAD - 1, dtype=jnp.int32)
    src_f = jnp.concatenate([src_r.reshape(TOT_CHUNKS, CHUNK), pad_sl])
    dst_f = jnp.concatenate([dst_r.reshape(TOT_CHUNKS, CHUNK), pad_sl])
    batch3 = jnp.pad(batch, (0, NPAD - N),
                     constant_values=NG).reshape(NBLK, 1, CHUNK)
    eye = jnp.eye(CHUNK, dtype=jnp.float32)
    zerosD = jnp.zeros((CHUNK, D), jnp.float32)
    b_embed2 = b_embed.reshape(1, D)
    b_gcn2 = b_gcn.reshape(1, D)
    b_mlp2 = b_mlp.reshape(1, D)

    dega = _dega_kernel(dst_r, eye, zerosD)
    y, dinvb = _tc_embed(x_pad, W_embed, W_gcn, b_embed2, dega)
    part = _scat_kernel(y, src_f, dst_f, zerosD)
    return _tc_final(part, y, dinvb, batch3, b_gcn2, W_mlp, b_mlp2)


# speedup vs baseline: 1.0005x; 1.0005x over previous
import functools

import jax
import jax.numpy as jnp
from jax import lax
from jax.experimental import pallas as pl
from jax.experimental.pallas import tpu as pltpu
from jax.experimental.pallas import tpu_sc as plsc

N = 10000
E = 320000
D = 128
NG = 16
NC, NS = 2, 16
NW = NC * NS
CHUNK = 128
CPW = 80
EPAD = NW * CPW * CHUNK
NPAD = 10112
ROWS_PT = NPAD // NS
NBLK = NPAD // CHUNK

_MESH = plsc.VectorSubcoreMesh(core_axis_name="c", subcore_axis_name="s",
                               num_cores=NC, num_subcores=NS)


def _zero_my_slice(acc, zeros_hbm, sid):
    base = sid * ROWS_PT
    nfull = ROWS_PT // CHUNK
    for k in range(nfull):
        pltpu.sync_copy(zeros_hbm, acc.at[pl.ds(base + k * CHUNK, CHUNK)])
    rem = ROWS_PT % CHUNK
    if rem:
        pltpu.sync_copy(zeros_hbm.at[pl.ds(0, rem)],
                        acc.at[pl.ds(base + nfull * CHUNK, rem)])


@functools.partial(
    pl.kernel,
    out_type=jax.ShapeDtypeStruct((NC, NPAD, D), jnp.float32),
    mesh=_MESH,
    scratch_types=[
        pltpu.VMEM_SHARED((NPAD, D), jnp.float32),
        pltpu.VMEM((CPW, CHUNK), jnp.int32),
        pltpu.VMEM((CHUNK, D), jnp.float32),
        pltpu.SemaphoreType.DMA,
    ],
)
def _dega_kernel(dst_hbm, eye_hbm, zeros_hbm, out_hbm, acc, dst_sl, eye_v, ssem):
    cid = lax.axis_index("c")
    sid = lax.axis_index("s")
    wid = sid * NC + cid
    _zero_my_slice(acc, zeros_hbm, sid)
    pltpu.sync_copy(eye_hbm, eye_v)
    pltpu.sync_copy(dst_hbm.at[wid], dst_sl)
    plsc.subcore_barrier()

    def body(j, carry):
        pltpu.async_copy(eye_v, acc.at[dst_sl.at[j]], ssem, add=True)

        @pl.when(j >= 4)
        def _():
            pltpu.make_async_copy(eye_v, acc.at[dst_sl.at[0]], ssem).wait()

        return carry

    lax.fori_loop(0, CPW, body, 0)
    for _ in range(4):
        pltpu.make_async_copy(eye_v, acc.at[dst_sl.at[0]], ssem).wait()
    plsc.subcore_barrier()
    pltpu.sync_copy(acc.at[pl.ds(sid * ROWS_PT, ROWS_PT)],
                    out_hbm.at[cid, pl.ds(sid * ROWS_PT, ROWS_PT)])


TOT_CHUNKS = EPAD // CHUNK
SLAB = CPW // 2


@functools.partial(
    pl.kernel,
    out_type=jax.ShapeDtypeStruct((NC, NPAD, D), jnp.float32),
    mesh=_MESH,
    scratch_types=[
        pltpu.VMEM_SHARED((NPAD, D), jnp.float32),
        pltpu.VMEM((SLAB, CHUNK), jnp.int32),
        pltpu.VMEM((SLAB, CHUNK), jnp.int32),
        [pltpu.VMEM((CHUNK, D), jnp.float32) for _ in range(2)],
        [pltpu.SemaphoreType.DMA for _ in range(2)],
        [pltpu.SemaphoreType.DMA for _ in range(2)],
    ],
)
def _scat_kernel(y_hbm, src_hbm, dst_hbm, zeros_hbm, out_hbm, acc,
                 src_sl, dst_sl, rows, gsem, ssem):
    cid = lax.axis_index("c")
    sid = lax.axis_index("s")
    _zero_my_slice(acc, zeros_hbm, sid)
    plsc.subcore_barrier()

    wid = sid * NC + cid
    base = wid * CPW

    for h in range(2):
        off = pl.multiple_of(base + h * SLAB, 8)
        pltpu.sync_copy(src_hbm.at[pl.ds(off, SLAB)], src_sl)
        pltpu.sync_copy(dst_hbm.at[pl.ds(off, SLAB)], dst_sl)

        def round_body(r, carry):
            for b in range(2):
                j = r * 2 + b

                @pl.when(r > 0)
                def _():
                    pltpu.make_async_copy(rows[b], acc.at[dst_sl.at[j]],
                                          ssem[b]).wait()

                pltpu.async_copy(y_hbm.at[src_sl.at[j]], rows[b], gsem[b])
            for b in range(2):
                j = r * 2 + b
                pltpu.make_async_copy(y_hbm.at[src_sl.at[j]], rows[b],
                                      gsem[b]).wait()
                pltpu.async_copy(rows[b], acc.at[dst_sl.at[j]], ssem[b],
                                 add=True)
            return carry

        lax.fori_loop(0, SLAB // 2, round_body, 0)
        for b in range(2):
            pltpu.make_async_copy(rows[b], acc.at[dst_sl.at[0]],
                                  ssem[b]).wait()
    plsc.subcore_barrier()
    pltpu.sync_copy(acc.at[pl.ds(sid * ROWS_PT, ROWS_PT)],
                    out_hbm.at[cid, pl.ds(sid * ROWS_PT, ROWS_PT)])


def _embed_body(x_ref, we_ref, wg_ref, be_ref, dega_ref, y_ref, dinv_ref):
    w2 = jnp.dot(we_ref[...], wg_ref[...], preferred_element_type=jnp.float32)
    c = jnp.dot(be_ref[...], wg_ref[...], preferred_element_type=jnp.float32)
    hw = jnp.dot(x_ref[...], w2, preferred_element_type=jnp.float32) + c
    deg = jnp.sum(dega_ref[0] + dega_ref[1], axis=1, keepdims=True) + 1.0
    dinv = jnp.broadcast_to(lax.rsqrt(deg), hw.shape)
    y_ref[...] = hw * dinv
    dinv_ref[...] = dinv


def _tc_embed(x_pad, W_embed, W_gcn, b_embed2, dega):
    return pl.pallas_call(
        _embed_body,
        out_shape=(jax.ShapeDtypeStruct((NPAD, D), jnp.float32),
                   jax.ShapeDtypeStruct((NPAD, D), jnp.float32)),
        grid=(NBLK,),
        in_specs=[
            pl.BlockSpec((CHUNK, D), lambda i: (i, 0)),
            pl.BlockSpec((D, D), lambda i: (0, 0)),
            pl.BlockSpec((D, D), lambda i: (0, 0)),
            pl.BlockSpec((1, D), lambda i: (0, 0)),
            pl.BlockSpec((NC, CHUNK, D), lambda i: (0, i, 0)),
        ],
        out_specs=(pl.BlockSpec((CHUNK, D), lambda i: (i, 0)),
                   pl.BlockSpec((CHUNK, D), lambda i: (i, 0))),
    )(x_pad, W_embed, W_gcn, b_embed2, dega)


def _final_body(part_ref, y_ref, dinv_ref, batch_ref, bg_ref, wm_ref, bm_ref,
                out_ref, gsum_ref, cnt_ref):
    i = pl.program_id(0)

    @pl.when(i == 0)
    def _():
        gsum_ref[...] = jnp.zeros_like(gsum_ref)
        cnt_ref[...] = jnp.zeros_like(cnt_ref)

    t = part_ref[0] + part_ref[1] + y_ref[...]
    h2 = jnp.maximum(t * dinv_ref[...] + bg_ref[...], 0.0)
    b = batch_ref[0]
    onehot_t = (lax.broadcasted_iota(jnp.int32, (NG, CHUNK), 0) == b
                ).astype(jnp.float32)
    gsum_ref[...] += jnp.dot(onehot_t, h2, preferred_element_type=jnp.float32)
    cnt_ref[...] = cnt_ref[...] + jnp.sum(onehot_t, axis=1, keepdims=True)

    @pl.when(i == NBLK - 1)
    def _():
        gf = gsum_ref[...] / jnp.maximum(cnt_ref[...], 1.0)
        out_ref[...] = jnp.dot(gf, wm_ref[...],
                               preferred_element_type=jnp.float32) + bm_ref[...]


def _tc_final(part, y, dinvb, batch3, b_gcn2, W_mlp, b_mlp2):
    return pl.pallas_call(
        _final_body,
        out_shape=jax.ShapeDtypeStruct((NG, D), jnp.float32),
        grid=(NBLK,),
        in_specs=[
            pl.BlockSpec((NC, CHUNK, D), lambda i: (0, i, 0)),
            pl.BlockSpec((CHUNK, D), lambda i: (i, 0)),
            pl.BlockSpec((CHUNK, D), lambda i: (i, 0)),
            pl.BlockSpec((1, 1, CHUNK), lambda i: (i, 0, 0)),
            pl.BlockSpec((1, D), lambda i: (0, 0)),
            pl.BlockSpec((D, D), lambda i: (0, 0)),
            pl.BlockSpec((1, D), lambda i: (0, 0)),
        ],
        out_specs=pl.BlockSpec((NG, D), lambda i: (0, 0)),
        scratch_shapes=[
            pltpu.VMEM((NG, D), jnp.float32),
            pltpu.VMEM((NG, 1), jnp.float32),
        ],
    )(part, y, dinvb, batch3, b_gcn2, W_mlp, b_mlp2)


def kernel(x, edge_index, batch, W_embed, b_embed, W_gcn, b_gcn,
           W_aff, b_aff, W_mlp, b_mlp):
    del W_aff, b_aff
    x_pad = jnp.pad(x, ((0, NPAD - N), (0, 0)))
    pad_idx = jnp.full((EPAD - E,), NPAD - 1, dtype=jnp.int32)
    src_r = jnp.concatenate([edge_index[0], pad_idx]).reshape(NW, CPW, CHUNK)
    dst_r = jnp.concatenate([edge_index[1], pad_idx]).reshape(NW, CPW, CHUNK)
    pad_sl = jnp.full((SLAB, CHUNK), NPAD - 1, dtype=jnp.int32)
    src_f = jnp.concatenate([src_r.reshape(TOT_CHUNKS, CHUNK), pad_sl])
    dst_f = jnp.concatenate([dst_r.reshape(TOT_CHUNKS, CHUNK), pad_sl])
    batch3 = jnp.pad(batch, (0, NPAD - N),
                     constant_values=NG).reshape(NBLK, 1, CHUNK)
    eye = jnp.eye(CHUNK, dtype=jnp.float32)
    zerosD = jnp.zeros((CHUNK, D), jnp.float32)
    b_embed2 = b_embed.reshape(1, D)
    b_gcn2 = b_gcn.reshape(1, D)
    b_mlp2 = b_mlp.reshape(1, D)

    dega = _dega_kernel(dst_r, eye, zerosD)
    y, dinvb = _tc_embed(x_pad, W_embed, W_gcn, b_embed2, dega)
    part = _scat_kernel(y, src_f, dst_f, zerosD)
    return _tc_final(part, y, dinvb, batch3, b_gcn2, W_mlp, b_mlp2)
